# baseline (speedup 1.0000x reference)
import functools

import jax
import jax.numpy as jnp
from jax import lax
from jax.experimental import pallas as pl
from jax.experimental.pallas import tpu as pltpu
from jax.experimental.pallas import tpu_sc as plsc

B, D, HW = 8, 256, 1024
N, M, DS = 32, 512, 8
BG = 4


def _tc_body_cbt(z_ref, cb_ref, idx_ref, cbT_ref=None):
    x = z_ref[0]
    for n in range(N):
        xs = x[n * DS:(n + 1) * DS, :]
        cbn = cb_ref[n]
        xc2 = lax.dot_general(cbn + cbn, xs, (((1,), (0,)), ((), ())),
                              preferred_element_type=jnp.float32)
        c2 = jnp.sum(cbn * cbn, axis=1, keepdims=True)
        x2 = jnp.sum(xs * xs, axis=0, keepdims=True)
        d = (x2 + c2) - xc2
        idx_ref[0, n, :] = jnp.argmin(d, axis=0)

    if cbT_ref is not None:
        @pl.when(pl.program_id(0) == 0)
        def _():
            for n in range(N):
                cbT_ref[n] = jnp.transpose(cb_ref[n], (1, 0))


def _tc_body(z_ref, cb_ref, idx_ref):
    _tc_body_cbt(z_ref, cb_ref, idx_ref)


def _tc_call(z3, cb, b0):
    with_cbt = b0 == 0
    out_specs = [pl.BlockSpec((1, N, HW), lambda b: (b, 0, 0))]
    out_shape = [jax.ShapeDtypeStruct((BG, N, HW), jnp.int32)]
    if with_cbt:
        out_specs.append(pl.BlockSpec((N, DS, M), lambda b: (0, 0, 0)))
        out_shape.append(jax.ShapeDtypeStruct((N, DS, M), jnp.float32))
    return pl.pallas_call(
        _tc_body_cbt if with_cbt else _tc_body,
        grid=(BG,),
        in_specs=[
            pl.BlockSpec((1, D, HW), lambda b: (b + b0, 0, 0)),
            pl.BlockSpec((N, M, DS), lambda b: (0, 0, 0)),
        ],
        out_specs=out_specs,
        out_shape=out_shape,
    )(z3, cb)


@functools.cache
def _sc_gather_build(b0):
    mesh = plsc.VectorSubcoreMesh(core_axis_name="c", subcore_axis_name="s")

    @functools.partial(
        pl.kernel,
        mesh=mesh,
        out_type=(
            jax.ShapeDtypeStruct((BG, D, HW), jnp.float32),
            jax.ShapeDtypeStruct((N, 16), jnp.float32),
        ),
        scratch_types=[
            pltpu.VMEM((DS * M,), jnp.float32),
            pltpu.VMEM((2, HW), jnp.int32),
            pltpu.VMEM((2, DS, HW), jnp.float32),
            pltpu.VMEM((2, DS, HW), jnp.float32),
            pltpu.VMEM((16,), jnp.float32),
            pltpu.SemaphoreType.DMA,
            pltpu.SemaphoreType.DMA,
            pltpu.SemaphoreType.DMA,
            pltpu.SemaphoreType.DMA,
            pltpu.SemaphoreType.DMA,
            pltpu.SemaphoreType.DMA,
        ],
        compiler_params=pltpu.CompilerParams(needs_layout_passes=False),
    )
    def sc_gather(cbT_hbm, idx_hbm, z_hbm, zq_hbm, loss_hbm,
                  tab_v, idx_v, xs_v, out_v, loss_v,
                  si0, si1, sx0, sx1, so0, so1):
        cid = lax.axis_index("c")
        sid = lax.axis_index("s")
        n = sid * 2 + cid
        sin = (si0, si1)
        sxs = (sx0, sx1)
        sout = (so0, so1)
        pltpu.sync_copy(cbT_hbm.at[n], tab_v)

        def start_in(b):
            p = b % 2
            h_i = pltpu.async_copy(idx_hbm.at[b, n], idx_v.at[p], sin[p])
            h_x = pltpu.async_copy(z_hbm.at[b + b0, pl.ds(n * DS, DS), :],
                                   xs_v.at[p], sxs[p])
            return h_i, h_x

        acc = jnp.zeros((16,), jnp.float32)
        pending_in = start_in(0)
        pending_out = [None, None]
        for b in range(BG):
            p = b % 2
            pending_in[0].wait()
            pending_in[1].wait()
            if b + 1 < BG:
                pending_in = start_in(b + 1)
            if pending_out[p] is not None:
                pending_out[p].wait()

            def body(i, carry):
                iv = idx_v[p, pl.ds(i * 16, 16)]
                for j in range(DS):
                    g = plsc.load_gather(tab_v, [iv + jnp.int32(j * M)])
                    out_v[p, j, pl.ds(i * 16, 16)] = g
                    df = xs_v[p, j, pl.ds(i * 16, 16)] - g
                    carry = carry + df * df
                return carry

            acc = lax.fori_loop(0, HW // 16, body, acc)
            pending_out[p] = pltpu.async_copy(
                out_v.at[p], zq_hbm.at[b, pl.ds(n * DS, DS), :], sout[p])
        loss_v[...] = acc
        pending_out[0].wait()
        if pending_out[1] is not None:
            pending_out[1].wait()
        pltpu.sync_copy(loss_v, loss_hbm.at[n])

    return sc_gather


def kernel(z, cb):
    z3 = z.reshape(B, D, HW)
    idx0, cbT3 = _tc_call(z3, cb, 0)
    idx1, = _tc_call(z3, cb, BG)
    cbT = cbT3.reshape(N, DS * M)
    zq0, l0 = _sc_gather_build(0)(cbT, idx0, z3)
    zq1, l1 = _sc_gather_build(BG)(cbT, idx1, z3)
    vq = (jnp.sum(l0) + jnp.sum(l1)) * jnp.float32(1.25 / (N * B * HW * DS))
    z_q = jnp.concatenate([zq0, zq1], axis=0).reshape(z.shape)
    indices = jnp.concatenate([idx0, idx1], axis=0).reshape(B, N, 32, 32)
    return (z_q, vq, indices)


# speedup vs baseline: 1.0944x; 1.0944x over previous
import functools

import jax
import jax.numpy as jnp
from jax import lax
from jax.experimental import pallas as pl
from jax.experimental.pallas import tpu as pltpu
from jax.experimental.pallas import tpu_sc as plsc

B, D, HW = 8, 256, 1024
N, M, DS = 32, 512, 8


def _tc_body(z_ref, cb_ref, idx_ref, cbT_ref):
    x = z_ref[0]
    for n in range(N):
        xs = x[n * DS:(n + 1) * DS, :]
        cbn = cb_ref[n]
        xc2 = lax.dot_general(cbn + cbn, xs, (((1,), (0,)), ((), ())),
                              preferred_element_type=jnp.float32)
        c2 = jnp.sum(cbn * cbn, axis=1, keepdims=True)
        x2 = jnp.sum(xs * xs, axis=0, keepdims=True)
        d = (x2 + c2) - xc2
        idx_ref[0, n, :] = jnp.argmin(d, axis=0)

    @pl.when(pl.program_id(0) == 0)
    def _():
        for n in range(N):
            cbT_ref[n] = jnp.transpose(cb_ref[n], (1, 0))


def _tc_call(z3, cb):
    return pl.pallas_call(
        _tc_body,
        grid=(B,),
        in_specs=[
            pl.BlockSpec((1, D, HW), lambda b: (b, 0, 0)),
            pl.BlockSpec((N, M, DS), lambda b: (0, 0, 0)),
        ],
        out_specs=[
            pl.BlockSpec((1, N, HW), lambda b: (b, 0, 0)),
            pl.BlockSpec((N, DS, M), lambda b: (0, 0, 0)),
        ],
        out_shape=[
            jax.ShapeDtypeStruct((B, N, HW), jnp.int32),
            jax.ShapeDtypeStruct((N, DS, M), jnp.float32),
        ],
    )(z3, cb)


@functools.cache
def _sc_gather_build():
    mesh = plsc.VectorSubcoreMesh(core_axis_name="c", subcore_axis_name="s")

    @functools.partial(
        pl.kernel,
        mesh=mesh,
        out_type=(
            jax.ShapeDtypeStruct((B, D, HW), jnp.float32),
            jax.ShapeDtypeStruct((N, 16), jnp.float32),
        ),
        scratch_types=[
            pltpu.VMEM((DS * M,), jnp.float32),
            pltpu.VMEM((2, HW), jnp.int32),
            pltpu.VMEM((2, DS, HW), jnp.float32),
            pltpu.VMEM((2, DS, HW), jnp.float32),
            pltpu.VMEM((16,), jnp.float32),
            pltpu.SemaphoreType.DMA,
            pltpu.SemaphoreType.DMA,
            pltpu.SemaphoreType.DMA,
            pltpu.SemaphoreType.DMA,
            pltpu.SemaphoreType.DMA,
            pltpu.SemaphoreType.DMA,
        ],
        compiler_params=pltpu.CompilerParams(needs_layout_passes=False),
    )
    def sc_gather(cbT_hbm, idx_hbm, z_hbm, zq_hbm, loss_hbm,
                  tab_v, idx_v, xs_v, out_v, loss_v,
                  si0, si1, sx0, sx1, so0, so1):
        cid = lax.axis_index("c")
        sid = lax.axis_index("s")
        n = sid * 2 + cid
        sin = (si0, si1)
        sxs = (sx0, sx1)
        sout = (so0, so1)
        pltpu.sync_copy(cbT_hbm.at[n], tab_v)

        def start_in(b):
            p = b % 2
            h_i = pltpu.async_copy(idx_hbm.at[b, n], idx_v.at[p], sin[p])
            h_x = pltpu.async_copy(z_hbm.at[b, pl.ds(n * DS, DS), :],
                                   xs_v.at[p], sxs[p])
            return h_i, h_x

        acc = tuple(jnp.zeros((16,), jnp.float32) for _ in range(DS))
        pending_in = start_in(0)
        pending_out = [None, None]
        for b in range(B):
            p = b % 2
            pending_in[0].wait()
            pending_in[1].wait()
            if b + 1 < B:
                pending_in = start_in(b + 1)
            if pending_out[p] is not None:
                pending_out[p].wait()

            def body(i, carry):
                iv = idx_v[p, pl.ds(i * 16, 16)]
                out = []
                for j in range(DS):
                    g = plsc.load_gather(tab_v, [iv + jnp.int32(j * M)])
                    out_v[p, j, pl.ds(i * 16, 16)] = g
                    df = xs_v[p, j, pl.ds(i * 16, 16)] - g
                    out.append(carry[j] + df * df)
                return tuple(out)

            acc = lax.fori_loop(0, HW // 16, body, acc)
            pending_out[p] = pltpu.async_copy(
                out_v.at[p], zq_hbm.at[b, pl.ds(n * DS, DS), :], sout[p])
        total = acc[0]
        for j in range(1, DS):
            total = total + acc[j]
        loss_v[...] = total
        pending_out[0].wait()
        pending_out[1].wait()
        pltpu.sync_copy(loss_v, loss_hbm.at[n])

    return sc_gather


def kernel(z, cb):
    z3 = z.reshape(B, D, HW)
    idx, cbT3 = _tc_call(z3, cb)
    cbT = cbT3.reshape(N, DS * M)
    zq3, loss_parts = _sc_gather_build()(cbT, idx, z3)
    vq = jnp.sum(loss_parts) * jnp.float32(1.25 / (N * B * HW * DS))
    z_q = zq3.reshape(z.shape)
    indices = idx.reshape(B, N, 32, 32)
    return (z_q, vq, indices)
